# baseline (speedup 1.0000x reference)
import functools

import jax
import jax.numpy as jnp
from jax import lax
from jax.experimental import pallas as pl
from jax.experimental.pallas import tpu as pltpu
from jax.experimental.pallas import tpu_sc as plsc

S = 512
K = 24
EPS = 1e-5


def _fps_body(c_ref, out_ref):
    B = out_ref.shape[0]
    N = c_ref.shape[1]
    n_iota = lax.broadcasted_iota(jnp.int32, (B, N), 1)
    s_iota = lax.broadcasted_iota(jnp.int32, (B, S), 1)
    n_iota3 = lax.broadcasted_iota(jnp.int32, (3 * B, N), 1)
    call = c_ref[...]

    def body(i, carry):
        dist, far = carry
        out_ref[...] = out_ref[...] + (s_iota == i).astype(jnp.int32) * far
        far3 = jnp.concatenate([far, far, far], axis=0)
        sel3 = n_iota3 == far3
        csum = jnp.sum(jnp.where(sel3, call, 0.0), axis=1,
                       keepdims=True)
        diff = call - csum
        sq = diff * diff
        d = sq[0:B, :] + sq[B:2 * B, :] + sq[2 * B:3 * B, :]
        dist = jnp.minimum(dist, d)
        m = jnp.max(dist, axis=1, keepdims=True)
        cand = jnp.where(dist == m, n_iota, N)
        far = jnp.min(cand, axis=1, keepdims=True)
        return dist, far

    dist0 = jnp.full((B, N), 1e10, dtype=jnp.float32)
    far0 = jnp.zeros((B, 1), dtype=jnp.int32)
    out_ref[...] = jnp.zeros((B, S), dtype=jnp.int32)
    lax.fori_loop(0, S, body, (dist0, far0))


def _fps(coords):
    B, N, _ = coords.shape
    c = jnp.transpose(coords, (2, 0, 1)).reshape(3 * B, N)
    return pl.pallas_call(
        _fps_body,
        out_shape=jax.ShapeDtypeStruct((B, S), jnp.int32),
    )(c)


_BS = 128


def _knn_body(d_ref, out_ref):
    e = d_ref[0]
    BS, N = e.shape

    n_iota = lax.broadcasted_iota(jnp.int32, (BS, N), 1)
    k_iota = lax.broadcasted_iota(jnp.int32, (BS, 128), 1)

    def body(t, carry):
        e, acc = carry
        m = jnp.min(e, axis=1, keepdims=True)
        amin = jnp.min(jnp.where(e == m, n_iota, N), axis=1,
                       keepdims=True)
        acc = jnp.where(k_iota == t, amin, acc)
        e = jnp.where(n_iota == amin, jnp.inf, e)
        return e, acc

    acc0 = jnp.zeros((BS, 128), dtype=jnp.int32)
    _, acc = lax.fori_loop(0, K, body, (e, acc0))
    out_ref[0] = acc


def _knn(d):
    B, _, N = d.shape
    out = pl.pallas_call(
        _knn_body,
        grid=(B, S // _BS),
        in_specs=[pl.BlockSpec((1, _BS, N), lambda b, sb: (b, sb, 0))],
        out_specs=pl.BlockSpec((1, _BS, 128), lambda b, sb: (b, sb, 0)),
        out_shape=jax.ShapeDtypeStruct((B, S, 128), jnp.int32),
    )(d)
    return out[:, :, :K]


def _make_seg_reduce(M, C, NW):
    nseg = M // NW
    CH = C // 16
    mesh = plsc.VectorSubcoreMesh(core_axis_name="c", subcore_axis_name="s")

    @functools.partial(
        pl.kernel, mesh=mesh,
        out_type=[
            jax.ShapeDtypeStruct((M, C), jnp.float32),
            jax.ShapeDtypeStruct((M, C), jnp.float32),
            jax.ShapeDtypeStruct((M, C), jnp.float32),
            jax.ShapeDtypeStruct((M, C), jnp.float32),
        ],
        scratch_types=[
            pltpu.VMEM((nseg * K,), jnp.int32),
            pltpu.VMEM((nseg,), jnp.int32),
            pltpu.VMEM((K, C), jnp.float32),
            pltpu.VMEM((nseg, C), jnp.float32),
            pltpu.VMEM((nseg, C), jnp.float32),
            pltpu.VMEM((nseg, C), jnp.float32),
            pltpu.VMEM((nseg, C), jnp.float32),
            pltpu.SemaphoreType.DMA,
        ],
    )
    def seg_reduce(ya_hbm, yc_hbm, idx_hbm, fps_hbm,
                   a1_hbm, a2_hbm, amax_hbm, z_hbm,
                   idx_v, fps_v, rows_v, a1_v, a2_v, amax_v, z_v, sem):
        wid = lax.axis_index("s") * 2 + lax.axis_index("c")
        base = wid * nseg
        pltpu.sync_copy(idx_hbm.at[pl.ds(base * K, nseg * K)], idx_v)
        pltpu.sync_copy(fps_hbm.at[pl.ds(base, nseg)], fps_v)
        pltpu.async_copy(yc_hbm.at[fps_v], z_v, sem).wait()

        def seg_body(j, _):
            pltpu.async_copy(
                ya_hbm.at[idx_v.at[pl.ds(j * K, K)]], rows_v, sem).wait()
            for h in range(CH):
                sl = pl.ds(h * 16, 16)
                r = rows_v[0, sl]
                s1 = r
                s2 = r * r
                mx = r
                for k in range(1, K):
                    r = rows_v[k, sl]
                    s1 = s1 + r
                    s2 = s2 + r * r
                    mx = jnp.maximum(mx, r)
                a1_v[j, sl] = s1
                a2_v[j, sl] = s2
                amax_v[j, sl] = mx
            return 0

        lax.fori_loop(0, nseg, seg_body, 0)
        pltpu.sync_copy(a1_v, a1_hbm.at[pl.ds(base, nseg)])
        pltpu.sync_copy(a2_v, a2_hbm.at[pl.ds(base, nseg)])
        pltpu.sync_copy(amax_v, amax_hbm.at[pl.ds(base, nseg)])
        pltpu.sync_copy(z_v, z_hbm.at[pl.ds(base, nseg)])

    return seg_reduce


def kernel(x, coords, W1, gamma1, beta1):
    B, D, N = x.shape
    C = W1.shape[0]
    feats = jnp.transpose(x, (0, 2, 1))

    fps = _fps(coords)

    W1a = W1[:, :D]
    W1c = W1[:, D:] - W1a
    Ya = jnp.einsum('bnd,cd->bnc', feats, W1a)
    Yc = jnp.einsum('bnd,cd->bnc', feats, W1c)

    new_xyz = jnp.take_along_axis(coords, fps[..., None], axis=1)
    d = (jnp.sum(new_xyz ** 2, -1, keepdims=True)
         - 2.0 * jnp.einsum('bsc,bnc->bsn', new_xyz, coords)
         + jnp.sum(coords ** 2, -1)[:, None, :])
    idx = _knn(d)

    boff = (jnp.arange(B, dtype=jnp.int32) * N)
    gidx = (idx + boff[:, None, None]).reshape(B * S * K)
    gfps = (fps + boff[:, None]).reshape(B * S)
    M = B * S
    A1, A2, Amax, Z = _make_seg_reduce(M, C, 32)(
        Ya.reshape(B * N, C), Yc.reshape(B * N, C), gidx, gfps)
    A1 = A1.reshape(B, S, C)
    A2 = A2.reshape(B, S, C)
    Amax = Amax.reshape(B, S, C)
    Z = Z.reshape(B, S, C)

    MK = B * S * K
    s1 = jnp.sum(A1 + K * Z, axis=(0, 1))
    s2 = jnp.sum(A2 + 2.0 * Z * A1 + K * Z * Z, axis=(0, 1))
    mean = s1 / MK
    var = s2 / MK - mean * mean

    inv = gamma1 / jnp.sqrt(var + EPS)
    hmax = Amax + Z
    out = jnp.maximum(hmax * inv[None, None, :] + (beta1 - mean * inv)[None, None, :], 0.0)
    return jnp.transpose(out, (0, 2, 1))


# speedup vs baseline: 1.1553x; 1.1553x over previous
import functools

import jax
import jax.numpy as jnp
from jax import lax
from jax.experimental import pallas as pl
from jax.experimental.pallas import tpu as pltpu
from jax.experimental.pallas import tpu_sc as plsc

S = 512
K = 24
EPS = 1e-5


def _fps_body(c_ref, out_ref):
    B = out_ref.shape[0]
    N = c_ref.shape[1]
    n_iota = lax.broadcasted_iota(jnp.int32, (B, N), 1)
    s_iota = lax.broadcasted_iota(jnp.int32, (B, S), 1)
    n_iota3 = lax.broadcasted_iota(jnp.int32, (3 * B, N), 1)
    call = c_ref[...]

    def body(i, carry):
        dist, far = carry
        out_ref[...] = out_ref[...] + (s_iota == i).astype(jnp.int32) * far
        far3 = jnp.concatenate([far, far, far], axis=0)
        sel3 = n_iota3 == far3
        csum = jnp.sum(jnp.where(sel3, call, 0.0), axis=1,
                       keepdims=True)
        diff = call - csum
        sq = diff * diff
        d = sq[0:B, :] + sq[B:2 * B, :] + sq[2 * B:3 * B, :]
        dist = jnp.minimum(dist, d)
        m = jnp.max(dist, axis=1, keepdims=True)
        cand = jnp.where(dist == m, n_iota, N)
        far = jnp.min(cand, axis=1, keepdims=True)
        return dist, far

    dist0 = jnp.full((B, N), 1e10, dtype=jnp.float32)
    far0 = jnp.zeros((B, 1), dtype=jnp.int32)
    out_ref[...] = jnp.zeros((B, S), dtype=jnp.int32)
    lax.fori_loop(0, S, body, (dist0, far0))


def _fps(coords):
    B, N, _ = coords.shape
    c = jnp.transpose(coords, (2, 0, 1)).reshape(3 * B, N)
    return pl.pallas_call(
        _fps_body,
        out_shape=jax.ShapeDtypeStruct((B, S), jnp.int32),
    )(c)


_BS = 512


def _knn_body(d_ref, out_ref):
    e = d_ref[0]
    BS, N = e.shape

    n_iota = lax.broadcasted_iota(jnp.int32, (BS, N), 1)
    k_iota = lax.broadcasted_iota(jnp.int32, (BS, 128), 1)

    def body(t, carry):
        e, acc = carry
        m = jnp.min(e, axis=1, keepdims=True)
        amin = jnp.min(jnp.where(e == m, n_iota, N), axis=1,
                       keepdims=True)
        acc = jnp.where(k_iota == t, amin, acc)
        e = jnp.where(n_iota == amin, jnp.inf, e)
        return e, acc

    acc0 = jnp.zeros((BS, 128), dtype=jnp.int32)
    _, acc = lax.fori_loop(0, K, body, (e, acc0))
    out_ref[0] = acc


def _knn(d):
    B, _, N = d.shape
    out = pl.pallas_call(
        _knn_body,
        grid=(B, S // _BS),
        in_specs=[pl.BlockSpec((1, _BS, N), lambda b, sb: (b, sb, 0))],
        out_specs=pl.BlockSpec((1, _BS, 128), lambda b, sb: (b, sb, 0)),
        out_shape=jax.ShapeDtypeStruct((B, S, 128), jnp.int32),
    )(d)
    return out[:, :, :K]


def _make_seg_reduce(M, C, NW):
    nseg = M // NW
    CH = C // 16
    mesh = plsc.VectorSubcoreMesh(core_axis_name="c", subcore_axis_name="s")

    @functools.partial(
        pl.kernel, mesh=mesh,
        out_type=[
            jax.ShapeDtypeStruct((M, C), jnp.float32),
            jax.ShapeDtypeStruct((M, C), jnp.float32),
            jax.ShapeDtypeStruct((M, C), jnp.float32),
            jax.ShapeDtypeStruct((M, C), jnp.float32),
        ],
        scratch_types=[
            pltpu.VMEM((nseg * K,), jnp.int32),
            pltpu.VMEM((nseg,), jnp.int32),
            pltpu.VMEM((K, C), jnp.float32),
            pltpu.VMEM((nseg, C), jnp.float32),
            pltpu.VMEM((nseg, C), jnp.float32),
            pltpu.VMEM((nseg, C), jnp.float32),
            pltpu.VMEM((nseg, C), jnp.float32),
            pltpu.SemaphoreType.DMA,
        ],
    )
    def seg_reduce(ya_hbm, yc_hbm, idx_hbm, fps_hbm,
                   a1_hbm, a2_hbm, amax_hbm, z_hbm,
                   idx_v, fps_v, rows_v, a1_v, a2_v, amax_v, z_v, sem):
        wid = lax.axis_index("s") * 2 + lax.axis_index("c")
        base = wid * nseg
        pltpu.sync_copy(idx_hbm.at[pl.ds(base * K, nseg * K)], idx_v)
        pltpu.sync_copy(fps_hbm.at[pl.ds(base, nseg)], fps_v)
        pltpu.async_copy(yc_hbm.at[fps_v], z_v, sem).wait()

        def seg_body(j, _):
            pltpu.async_copy(
                ya_hbm.at[idx_v.at[pl.ds(j * K, K)]], rows_v, sem).wait()
            for h in range(CH):
                sl = pl.ds(h * 16, 16)
                r = rows_v[0, sl]
                s1 = r
                s2 = r * r
                mx = r
                for k in range(1, K):
                    r = rows_v[k, sl]
                    s1 = s1 + r
                    s2 = s2 + r * r
                    mx = jnp.maximum(mx, r)
                a1_v[j, sl] = s1
                a2_v[j, sl] = s2
                amax_v[j, sl] = mx
            return 0

        lax.fori_loop(0, nseg, seg_body, 0)
        pltpu.sync_copy(a1_v, a1_hbm.at[pl.ds(base, nseg)])
        pltpu.sync_copy(a2_v, a2_hbm.at[pl.ds(base, nseg)])
        pltpu.sync_copy(amax_v, amax_hbm.at[pl.ds(base, nseg)])
        pltpu.sync_copy(z_v, z_hbm.at[pl.ds(base, nseg)])

    return seg_reduce


def kernel(x, coords, W1, gamma1, beta1):
    B, D, N = x.shape
    C = W1.shape[0]
    feats = jnp.transpose(x, (0, 2, 1))

    fps = _fps(coords)

    W1a = W1[:, :D]
    W1c = W1[:, D:] - W1a
    Ya = jnp.einsum('bnd,cd->bnc', feats, W1a)
    Yc = jnp.einsum('bnd,cd->bnc', feats, W1c)

    new_xyz = jnp.take_along_axis(coords, fps[..., None], axis=1)
    d = (jnp.sum(new_xyz ** 2, -1, keepdims=True)
         - 2.0 * jnp.einsum('bsc,bnc->bsn', new_xyz, coords)
         + jnp.sum(coords ** 2, -1)[:, None, :])
    idx = _knn(d)

    boff = (jnp.arange(B, dtype=jnp.int32) * N)
    gidx = (idx + boff[:, None, None]).reshape(B * S * K)
    gfps = (fps + boff[:, None]).reshape(B * S)
    M = B * S
    A1, A2, Amax, Z = _make_seg_reduce(M, C, 32)(
        Ya.reshape(B * N, C), Yc.reshape(B * N, C), gidx, gfps)
    A1 = A1.reshape(B, S, C)
    A2 = A2.reshape(B, S, C)
    Amax = Amax.reshape(B, S, C)
    Z = Z.reshape(B, S, C)

    MK = B * S * K
    s1 = jnp.sum(A1 + K * Z, axis=(0, 1))
    s2 = jnp.sum(A2 + 2.0 * Z * A1 + K * Z * Z, axis=(0, 1))
    mean = s1 / MK
    var = s2 / MK - mean * mean

    inv = gamma1 / jnp.sqrt(var + EPS)
    hmax = Amax + Z
    out = jnp.maximum(hmax * inv[None, None, :] + (beta1 - mean * inv)[None, None, :], 0.0)
    return jnp.transpose(out, (0, 2, 1))
